# Initial kernel scaffold; baseline (speedup 1.0000x reference)
import jax
import jax.numpy as jnp
from jax.experimental import pallas as pl


def kernel(inputs, w_tables, v_tables, b, W1, b1, W2, b2, W3, b3):
    raise NotImplementedError("write your pallas kernel here")


# speedup vs baseline: 1.8938x; 1.8938x over previous
import functools

import jax
import jax.numpy as jnp
from jax import lax
from jax.experimental import pallas as pl
from jax.experimental.pallas import tpu as pltpu
from jax.experimental.pallas import tpu_sc as plsc

F = 26
V = 100000
E = 16
B = 16384
RV = V // 8
BC = 512
GRID = (RV + BC - 1) // BC

_NC, _NS = 2, 16
_NW = _NC * _NS
_BPW = B // _NW


def _table_body(vref, wref, w1ref, w2ref, w3ref, bref, b1ref, b2ref, b3ref,
                oref):
    we = jnp.dot(w2ref[...], w3ref[...], preferred_element_type=jnp.float32)
    weff = jnp.dot(w1ref[...], we, preferred_element_type=jnp.float32)
    c0 = (bref[...] * weff[442:443, :]
          + jnp.dot(b1ref[...], we, preferred_element_type=jnp.float32)
          + jnp.dot(b2ref[...], w3ref[...], preferred_element_type=jnp.float32)
          + b3ref[...])

    wlin = weff[0:F, :]
    acc = jnp.sum(wref[...] * wlin[:, :, None], axis=0)

    row = lax.broadcasted_iota(jnp.int32, (128, 8), 0)
    col = lax.broadcasted_iota(jnp.int32, (128, 8), 1)
    sel = (row // E == col).astype(jnp.float32)
    vb = vref[...]
    for f in range(F):
        wvf = weff[F + E * f:F + E * (f + 1), :]
        wt = jnp.concatenate([wvf] * 8, axis=0)
        acc = acc + jnp.dot(vb[f], sel * wt,
                            preferred_element_type=jnp.float32)
    oref[...] = jax.nn.sigmoid(acc + c0)


def _build_table(v3, w3, W1, W2, W3, b2d, b1r, b2r, b3r):
    return pl.pallas_call(
        _table_body,
        grid=(GRID,),
        in_specs=[
            pl.BlockSpec((F, BC, 128), lambda i: (0, i, 0)),
            pl.BlockSpec((F, BC, 8), lambda i: (0, i, 0)),
            pl.BlockSpec((443, 256), lambda i: (0, 0)),
            pl.BlockSpec((256, 128), lambda i: (0, 0)),
            pl.BlockSpec((128, 1), lambda i: (0, 0)),
            pl.BlockSpec((1, 1), lambda i: (0, 0)),
            pl.BlockSpec((1, 256), lambda i: (0, 0)),
            pl.BlockSpec((1, 128), lambda i: (0, 0)),
            pl.BlockSpec((1, 1), lambda i: (0, 0)),
        ],
        out_specs=pl.BlockSpec((BC, 8), lambda i: (i, 0)),
        out_shape=jax.ShapeDtypeStruct((RV, 8), jnp.float32),
    )(v3, w3, W1, W2, W3, b2d, b1r, b2r, b3r)


def _gather_body(s_hbm, idx_hbm, out_hbm, idx_v, rows_v, sem):
    wid = lax.axis_index("s") * _NC + lax.axis_index("c")
    base = wid * _BPW
    pltpu.sync_copy(idx_hbm.at[pl.ds(base, _BPW)], idx_v)
    pltpu.async_copy(s_hbm.at[idx_v], rows_v, sem).wait()
    pltpu.sync_copy(rows_v, out_hbm.at[pl.ds(base, _BPW)])


def _gather(s2, idx):
    mesh = plsc.VectorSubcoreMesh(core_axis_name="c", subcore_axis_name="s")
    run = functools.partial(
        pl.kernel,
        mesh=mesh,
        out_type=jax.ShapeDtypeStruct((B,), jnp.float32),
        scratch_types=[
            pltpu.VMEM((_BPW,), jnp.int32),
            pltpu.VMEM((_BPW,), jnp.float32),
            pltpu.SemaphoreType.DMA,
        ],
    )(_gather_body)
    return run(s2, idx)


def kernel(inputs, w_tables, v_tables, b, W1, b1, W2, b2, W3, b3):
    v3 = v_tables.reshape(F, RV, 128)
    w3 = w_tables.reshape(F, RV, 8)
    s = _build_table(v3, w3, W1, W2, W3,
                     b.reshape(1, 1), b1.reshape(1, 256),
                     b2.reshape(1, 128), b3.reshape(1, 1))
    return _gather(s.reshape(V), inputs).reshape(B, 1)
